# baseline (speedup 1.0000x reference)
import jax
import jax.numpy as jnp
from jax.experimental import pallas as pl

B = 64
BB = 16
N = 12
HID = 64


def _dot(a, b):
    return jax.lax.dot_general(
        a, b, (((1,), (0,)), ((), ())), preferred_element_type=jnp.float32)


def _rp(x):
    u = jax.lax.bitcast_convert_type(x, jnp.uint32)
    u = (u + jnp.uint32(0x7FFF) + ((u >> 16) & jnp.uint32(1))) & jnp.uint32(0xFFFF0000)
    return jax.lax.bitcast_convert_type(u, jnp.float32)


def _gat(xl, xr, att, bias, adj):
    v = xl[:, None, :, :] + xr[:, :, None, :]
    lr = jnp.where(v >= 0.0, v, 0.2 * v)
    e = jnp.sum(_rp(lr) * _rp(att)[None, None, None, :], axis=-1)
    e = jnp.where(adj, e, -1e9)
    e = e - jnp.max(e, axis=2, keepdims=True)
    ex = jnp.exp(e)
    a = ex / jnp.sum(ex, axis=2, keepdims=True)
    a = jnp.where(adj, a, 0.0)
    out = jnp.sum(_rp(a)[:, :, :, None] * _rp(xl)[:, None, :, :], axis=2)
    return out + bias[None, None, :]


def _fused_kernel(batch_ref,
                  w1, b1, w2, b2, w3, b3,
                  s1l, s1r, g1a, g1b,
                  g2l, g2r, g2a, g2b,
                  g3l, g3r, g3a, g3b,
                  g4l, g4r, g4a, g4b,
                  wlab, bias8, wv, ws, bs,
                  heads_ref, latent_ref, adj_ref):
    obs = batch_ref[...].reshape(BB * N, 5)
    h = jnp.maximum(_dot(obs, w1[...].T) + b1[...][None, :], 0.0)
    h = jnp.maximum(_dot(h, w2[...].T) + b2[...][None, :], 0.0)
    latent = _dot(h, w3[...].T) + b3[...][None, :]
    lat3 = latent.reshape(BB, N, 3)
    latent_ref[...] = lat3

    px = lat3[:, :, 0]
    py = lat3[:, :, 1]
    pxi = jnp.repeat(px, N, axis=1)
    pxj = jnp.tile(px, (1, N))
    pyi = jnp.repeat(py, N, axis=1)
    pyj = jnp.tile(py, (1, N))
    midx = (pxi + pxj) / 2.0
    midy = (pyi + pyj) / 2.0
    dx = pxi - pxj
    dy = pyi - pyj
    r2 = (dx * dx + dy * dy) / 4.0
    ddx = px[:, :, None] - midx[:, None, :]
    ddy = py[:, :, None] - midy[:, None, :]
    d2 = ddx * ddx + ddy * ddy
    kdx = jax.lax.broadcasted_iota(jnp.int32, (N, N * N), 0)
    ldx = jax.lax.broadcasted_iota(jnp.int32, (N, N * N), 1)
    idx = ldx // N
    jdx = ldx - idx * N
    excl = (kdx == idx) | (kdx == jdx)
    inside = (d2 < r2[:, None, :]) & (~excl)[None, :, :]
    eyel = (idx[0] == jdx[0])
    adjf = ((~jnp.any(inside, axis=1)) & (~eyel)[None, :]) | eyel[None, :]
    adj_ref[...] = adjf
    adj = adjf.astype(jnp.float32).reshape(BB, N, N) > 0.5

    x = lat3[:, :, 2:3]
    xl = x * s1l[...][None, :, :]
    xr = x * s1r[...][None, :, :]
    x1 = jnp.maximum(_gat(xl, xr, g1a[...], g1b[...], adj), 0.0)

    x1f = x1.reshape(BB * N, HID)
    xl = _dot(x1f, g2l[...].T).reshape(BB, N, HID)
    xr = _dot(x1f, g2r[...].T).reshape(BB, N, HID)
    x2 = jnp.maximum(_gat(xl, xr, g2a[...], g2b[...], adj), 0.0)

    skip = (_dot(latent, ws[...].T) + bs[...][None, :]).reshape(BB, N, HID)

    x2f = x2.reshape(BB * N, HID)
    xl = _dot(x2f, g3l[...].T).reshape(BB, N, HID)
    xr = _dot(x2f, g3r[...].T).reshape(BB, N, HID)
    x3 = jnp.maximum(_gat(xl, xr, g3a[...], g3b[...], adj) + 0.1 * skip, 0.0)

    xl = _dot(x2f, g4l[...].T).reshape(BB, N, HID)
    xr = _dot(x2f, g4r[...].T).reshape(BB, N, HID)
    x4 = jnp.maximum(_gat(xl, xr, g4a[...], g4b[...], adj) + 0.1 * skip, 0.0)

    heads = (_dot(x3.reshape(BB * N, HID), wlab[...])
             + _dot(x4.reshape(BB * N, HID), wv[...])
             + bias8[...][None, :])
    heads_ref[...] = heads.reshape(BB, N, 8)


def _rep(shape):
    nd = len(shape)
    return pl.BlockSpec(shape, lambda i: (0,) * nd)


def kernel(batch, params):
    wlab, blab = params['label_head']
    wv, bv = params['value_head']
    wlab8 = jnp.zeros((HID, 8), jnp.float32).at[:, :4].set(wlab.T)
    wv8 = jnp.zeros((HID, 8), jnp.float32).at[:, 4:5].set(wv.T)
    bias8 = jnp.zeros((8,), jnp.float32).at[:4].set(blab).at[4].set(bv[0])
    g1l, g1r, g1a, g1b = params['gcn1']
    s1l = g1l.T
    s1r = g1r.T
    flat = [batch,
            *params['enc1'], *params['enc2'], *params['enc3'],
            s1l, s1r, g1a, g1b,
            *params['gcn2'], *params['gcn3'], *params['gcn4'],
            wlab8, bias8, wv8, *params['skip']]
    in_specs = [pl.BlockSpec((BB, N, 5), lambda i: (i, 0, 0))]
    in_specs += [_rep(a.shape) for a in flat[1:]]
    out_shapes = (
        jax.ShapeDtypeStruct((B, N, 8), jnp.float32),
        jax.ShapeDtypeStruct((B, N, 3), jnp.float32),
        jax.ShapeDtypeStruct((B, N * N), jnp.bool_),
    )
    out_specs = (
        pl.BlockSpec((BB, N, 8), lambda i: (i, 0, 0)),
        pl.BlockSpec((BB, N, 3), lambda i: (i, 0, 0)),
        pl.BlockSpec((BB, N * N), lambda i: (i, 0)),
    )
    heads, latent, adj = pl.pallas_call(
        _fused_kernel,
        grid=(B // BB,),
        in_specs=in_specs,
        out_specs=out_specs,
        out_shape=out_shapes,
    )(*flat)
    return (batch[:, :, :4], batch[:, :, 4].reshape(B, N, 1),
            heads[:, :, :4], heads[:, :, 4:5], latent, adj.reshape(B, N, N))


# speedup vs baseline: 1.1416x; 1.1416x over previous
import jax
import jax.numpy as jnp
from jax.experimental import pallas as pl

B = 64
BB = 64
N = 12
HID = 64


def _dot(a, b):
    return jax.lax.dot_general(
        a, b, (((1,), (0,)), ((), ())), preferred_element_type=jnp.float32)


def _rp(x):
    return x.astype(jnp.bfloat16).astype(jnp.float32)


def _gat(xl, xr, att, bias, adj):
    v = xl[:, None, :, :] + xr[:, :, None, :]
    lr = jnp.where(v >= 0.0, v, 0.2 * v)
    e = jnp.sum(_rp(lr) * _rp(att)[None, None, None, :], axis=-1)
    e = jnp.where(adj, e, -1e9)
    e = e - jnp.max(e, axis=2, keepdims=True)
    ex = jnp.exp(e)
    a = ex / jnp.sum(ex, axis=2, keepdims=True)
    a = jnp.where(adj, a, 0.0)
    out = jnp.sum(_rp(a)[:, :, :, None] * _rp(xl)[:, None, :, :], axis=2)
    return out + bias[None, None, :]


def _fused_kernel(batch_ref,
                  w1, b1, w2, b2, w3, b3,
                  s1l, s1r, g1a, g1b,
                  g2l, g2r, g2a, g2b,
                  g3l, g3r, g3a, g3b,
                  g4l, g4r, g4a, g4b,
                  wlab, bias8, wv, ws, bs,
                  heads_ref, latent_ref, adj_ref):
    obs = batch_ref[...].reshape(BB * N, 5)
    h = jnp.maximum(_dot(obs, w1[...].T) + b1[...][None, :], 0.0)
    h = jnp.maximum(_dot(h, w2[...].T) + b2[...][None, :], 0.0)
    latent = _dot(h, w3[...].T) + b3[...][None, :]
    lat3 = latent.reshape(BB, N, 3)
    latent_ref[...] = lat3

    px = lat3[:, :, 0]
    py = lat3[:, :, 1]
    pxi = jnp.repeat(px, N, axis=1)
    pxj = jnp.tile(px, (1, N))
    pyi = jnp.repeat(py, N, axis=1)
    pyj = jnp.tile(py, (1, N))
    midx = (pxi + pxj) / 2.0
    midy = (pyi + pyj) / 2.0
    dx = pxi - pxj
    dy = pyi - pyj
    r2 = (dx * dx + dy * dy) / 4.0
    ddx = px[:, :, None] - midx[:, None, :]
    ddy = py[:, :, None] - midy[:, None, :]
    d2 = ddx * ddx + ddy * ddy
    kdx = jax.lax.broadcasted_iota(jnp.int32, (N, N * N), 0)
    ldx = jax.lax.broadcasted_iota(jnp.int32, (N, N * N), 1)
    idx = ldx // N
    jdx = ldx - idx * N
    excl = (kdx == idx) | (kdx == jdx)
    inside = (d2 < r2[:, None, :]) & (~excl)[None, :, :]
    eyel = (idx[0] == jdx[0])
    adjf = ((~jnp.any(inside, axis=1)) & (~eyel)[None, :]) | eyel[None, :]
    adj_ref[...] = adjf
    adj = adjf.astype(jnp.float32).reshape(BB, N, N) > 0.5

    x = lat3[:, :, 2:3]
    xl = x * s1l[...][None, :, :]
    xr = x * s1r[...][None, :, :]
    x1 = jnp.maximum(_gat(xl, xr, g1a[...], g1b[...], adj), 0.0)

    x1f = x1.reshape(BB * N, HID)
    xl = _dot(x1f, g2l[...].T).reshape(BB, N, HID)
    xr = _dot(x1f, g2r[...].T).reshape(BB, N, HID)
    x2 = jnp.maximum(_gat(xl, xr, g2a[...], g2b[...], adj), 0.0)

    skip = (_dot(latent, ws[...].T) + bs[...][None, :]).reshape(BB, N, HID)

    x2f = x2.reshape(BB * N, HID)
    xl = _dot(x2f, g3l[...].T).reshape(BB, N, HID)
    xr = _dot(x2f, g3r[...].T).reshape(BB, N, HID)
    x3 = jnp.maximum(_gat(xl, xr, g3a[...], g3b[...], adj) + 0.1 * skip, 0.0)

    xl = _dot(x2f, g4l[...].T).reshape(BB, N, HID)
    xr = _dot(x2f, g4r[...].T).reshape(BB, N, HID)
    x4 = jnp.maximum(_gat(xl, xr, g4a[...], g4b[...], adj) + 0.1 * skip, 0.0)

    heads = (_dot(x3.reshape(BB * N, HID), wlab[...])
             + _dot(x4.reshape(BB * N, HID), wv[...])
             + bias8[...][None, :])
    heads_ref[...] = heads.reshape(BB, N, 8)


def _rep(shape):
    nd = len(shape)
    return pl.BlockSpec(shape, lambda i: (0,) * nd)


def kernel(batch, params):
    wlab, blab = params['label_head']
    wv, bv = params['value_head']
    wlab8 = jnp.zeros((HID, 8), jnp.float32).at[:, :4].set(wlab.T)
    wv8 = jnp.zeros((HID, 8), jnp.float32).at[:, 4:5].set(wv.T)
    bias8 = jnp.zeros((8,), jnp.float32).at[:4].set(blab).at[4].set(bv[0])
    g1l, g1r, g1a, g1b = params['gcn1']
    s1l = g1l.T
    s1r = g1r.T
    flat = [batch,
            *params['enc1'], *params['enc2'], *params['enc3'],
            s1l, s1r, g1a, g1b,
            *params['gcn2'], *params['gcn3'], *params['gcn4'],
            wlab8, bias8, wv8, *params['skip']]
    in_specs = [pl.BlockSpec((BB, N, 5), lambda i: (i, 0, 0))]
    in_specs += [_rep(a.shape) for a in flat[1:]]
    out_shapes = (
        jax.ShapeDtypeStruct((B, N, 8), jnp.float32),
        jax.ShapeDtypeStruct((B, N, 3), jnp.float32),
        jax.ShapeDtypeStruct((B, N * N), jnp.bool_),
    )
    out_specs = (
        pl.BlockSpec((BB, N, 8), lambda i: (i, 0, 0)),
        pl.BlockSpec((BB, N, 3), lambda i: (i, 0, 0)),
        pl.BlockSpec((BB, N * N), lambda i: (i, 0)),
    )
    heads, latent, adj = pl.pallas_call(
        _fused_kernel,
        grid=(B // BB,),
        in_specs=in_specs,
        out_specs=out_specs,
        out_shape=out_shapes,
    )(*flat)
    return (batch[:, :, :4], batch[:, :, 4].reshape(B, N, 1),
            heads[:, :, :4], heads[:, :, 4:5], latent, adj.reshape(B, N, N))
